# baseline (speedup 1.0000x reference)
import jax
import jax.numpy as jnp
from jax.experimental import pallas as pl
from jax.experimental.pallas import tpu as pltpu

_BM = 400


def _gcn_body(x_ref, adj_ref, w_ref, b_ref, o_ref):
    i = pl.program_id(0)
    d = x_ref.shape[1]
    agg = jnp.dot(adj_ref[...].astype(jnp.bfloat16),
                  x_ref[...].astype(jnp.bfloat16),
                  preferred_element_type=jnp.float32)
    xs = x_ref[pl.ds(i * _BM, _BM), :]
    h = (jnp.dot(xs, w_ref[0:d, :], preferred_element_type=jnp.float32)
         + jnp.dot(agg, w_ref[d:2 * d, :], preferred_element_type=jnp.float32)
         + b_ref[...])
    o_ref[...] = jnp.maximum(h, 0.0)


def kernel(x, adj, W, b):
    n, d = x.shape
    nh = W.shape[1]
    grid = (n // _BM,)
    return pl.pallas_call(
        _gcn_body,
        grid=grid,
        in_specs=[
            pl.BlockSpec((n, d), lambda i: (0, 0)),
            pl.BlockSpec((_BM, n), lambda i: (i, 0)),
            pl.BlockSpec((2 * d, nh), lambda i: (0, 0)),
            pl.BlockSpec((1, nh), lambda i: (0, 0)),
        ],
        out_specs=pl.BlockSpec((_BM, nh), lambda i: (i, 0)),
        out_shape=jax.ShapeDtypeStruct((n, nh), jnp.float32),
        compiler_params=pltpu.CompilerParams(
            dimension_semantics=("arbitrary",),
        ),
    )(x, adj, W, b.reshape(1, nh))


# speedup vs baseline: 1.0039x; 1.0039x over previous
import jax
import jax.numpy as jnp
from jax.experimental import pallas as pl
from jax.experimental.pallas import tpu as pltpu

_BM = 400


def _gcn_body(x_ref, adj_ref, w_ref, b_ref, o_ref):
    i = pl.program_id(0)
    d = x_ref.shape[1]
    agg = jnp.dot(adj_ref[...].astype(jnp.bfloat16),
                  x_ref[...].astype(jnp.bfloat16),
                  preferred_element_type=jnp.float32)
    xs = x_ref[pl.ds(i * _BM, _BM), :]
    h = (jnp.dot(xs, w_ref[0:d, :], preferred_element_type=jnp.float32)
         + jnp.dot(agg, w_ref[d:2 * d, :], preferred_element_type=jnp.float32)
         + b_ref[...])
    o_ref[...] = jnp.maximum(h, 0.0)


def kernel(x, adj, W, b):
    n, d = x.shape
    nh = W.shape[1]
    grid = (n // _BM,)
    return pl.pallas_call(
        _gcn_body,
        grid=grid,
        in_specs=[
            pl.BlockSpec((n, d), lambda i: (0, 0)),
            pl.BlockSpec((_BM, n), lambda i: (i, 0)),
            pl.BlockSpec((2 * d, nh), lambda i: (0, 0)),
            pl.BlockSpec((1, nh), lambda i: (0, 0)),
        ],
        out_specs=pl.BlockSpec((_BM, nh), lambda i: (i, 0)),
        out_shape=jax.ShapeDtypeStruct((n, nh), jnp.float32),
        compiler_params=pltpu.CompilerParams(
            dimension_semantics=("parallel",),
        ),
    )(x, adj, W, b.reshape(1, nh))
